# baseline (speedup 1.0000x reference)
import functools

import jax
import jax.numpy as jnp
import numpy as np
from jax import lax
from jax.experimental import pallas as pl
from jax.experimental.pallas import tpu as pltpu
from jax.experimental.pallas import tpu_sc as plsc

N = 10013
E = 320416
NPAD = 10880
NC, NS = 2, 16
NW = NC * NS
H = NPAD // 2
ROWS = NPAD // NS
BT = (E // NW + 127) // 128
BT1 = BT + 1
EPAD = NW * BT * 128
DOUTS = [64, 64, 64, 128, 128, 128, 256, 256, 256]
R_TC = 640
R_CD = 136

_mesh = plsc.VectorSubcoreMesh(core_axis_name="c", subcore_axis_name="s")


def _zero_fill(ref, rows, cols):
    z = jnp.zeros((16,), jnp.float32)
    per_row = cols // 16

    def body(t, carry):
        ref[t // per_row, pl.ds((t % per_row) * 16, 16)] = z
        return carry

    lax.fori_loop(0, rows * per_row, body, 0, unroll=8)


@functools.partial(
    pl.kernel,
    out_type=jax.ShapeDtypeStruct((NC, NPAD, 16), jnp.float32),
    mesh=_mesh,
    scratch_types=[
        pltpu.VMEM((BT, 128), jnp.int32),
        pltpu.VMEM((128, 16), jnp.float32),
        pltpu.VMEM((128, 16), jnp.float32),
        pltpu.VMEM((128,), jnp.int32),
        pltpu.VMEM_SHARED((NPAD, 16), jnp.float32),
    ],
)
def _sc_degree(dst_hbm, out_hbm, didx, obuf, zbuf, ibuf, acc):
    c = lax.axis_index("c")
    s = lax.axis_index("s")
    wid = s * NC + c
    base = s * ROWS
    pltpu.sync_copy(dst_hbm.at[wid], didx)
    one = jnp.ones((16,), jnp.float32)

    def fill(t, carry):
        obuf[t, pl.ds(0, 16)] = one
        return carry

    lax.fori_loop(0, 128, fill, 0, unroll=8)
    _zero_fill(zbuf, 128, 16)
    off = 0
    while off < ROWS:
        sz = min(128, ROWS - off)
        pltpu.sync_copy(zbuf.at[pl.ds(0, sz)], acc.at[pl.ds(base + off, sz)])
        off += sz
    plsc.subcore_barrier()

    def bat(b, carry):
        for j in range(8):
            ibuf[pl.ds(j * 16, 16)] = didx[b, pl.ds(j * 16, 16)]
        pltpu.sync_copy(obuf, acc.at[ibuf], add=True)
        return carry

    lax.fori_loop(0, BT, bat, 0)
    plsc.subcore_barrier()
    pltpu.sync_copy(acc.at[pl.ds(base, ROWS)], out_hbm.at[c, pl.ds(base, ROWS)])


def _acc_init(h_hbm, acc, c, s):
    @pl.when(s < NS - 1)
    def _():
        pltpu.sync_copy(h_hbm.at[pl.ds(c * H + s * 344, 344)],
                        acc.at[pl.ds(s * 344, 344)])

    @pl.when(s == NS - 1)
    def _():
        pltpu.sync_copy(h_hbm.at[pl.ds(c * H + 5160, 280)],
                        acc.at[pl.ds(5160, 280)])


def _acc_writeback(acc, out_hbm, c, s):
    @pl.when(s < NS - 1)
    def _():
        pltpu.sync_copy(acc.at[pl.ds(s * 344, 344)],
                        out_hbm.at[pl.ds(c * H + s * 344, 344)])

    @pl.when(s == NS - 1)
    def _():
        pltpu.sync_copy(acc.at[pl.ds(5160, 280)],
                        out_hbm.at[pl.ds(c * H + 5160, 280)])


HSPR = 64
SSPR = 512


def _make_sc_segment(K):
    scratch = [
        pltpu.VMEM((BT, 128), jnp.int32),
        pltpu.VMEM((BT, 128), jnp.int32),
        pltpu.VMEM((2, 128, 128), jnp.float32),
        pltpu.VMEM((128,), jnp.int32),
        pltpu.VMEM_SHARED((H + HSPR, 128), jnp.float32),
        pltpu.SemaphoreType.DMA,
    ]

    def body(src_hbm, dst_hbm, *rest):
        h_hbms = rest[:K]
        out_hbm = rest[K]
        sidx, didx, gbuf, ibuf, acc, gsem = rest[K + 1:]
        c = lax.axis_index("c")
        s = lax.axis_index("s")
        wid = s * NC + c
        pltpu.sync_copy(src_hbm.at[wid], sidx)
        pltpu.sync_copy(dst_hbm.at[wid], didx)
        iota16 = lax.iota(jnp.int32, 16)
        hvec = jnp.full((16,), H, jnp.int32)

        def make_rewrite(lo):
            def rewrite(t, carry):
                i = t // 8
                j = t % 8
                svec = sidx[i, pl.ds(j * 16, 16)]
                dvec = didx[i, pl.ds(j * 16, 16)]
                loc = dvec - lo
                inr = lax.shift_right_logical(loc - hvec, 31) * (
                    1 - lax.shift_right_logical(loc, 31))
                lane = t * 16 + iota16
                nsrc = N + (lane & (SSPR - 1))
                ndst = H + (lane & (HSPR - 1))
                out = 1 - inr
                sidx[i, pl.ds(j * 16, 16)] = inr * svec + out * nsrc
                didx[i, pl.ds(j * 16, 16)] = inr * loc + out * ndst
                return carry
            return rewrite

        @pl.when(c == 0)
        def _():
            lax.fori_loop(0, BT * 8, make_rewrite(0), 0)

        @pl.when(c == 1)
        def _():
            lax.fori_loop(0, BT * 8, make_rewrite(H), 0)

        for k in range(K):
            _acc_init(h_hbms[k], acc, c, s)
            plsc.subcore_barrier()
            def bat(b, carry, k=k):
                pltpu.sync_copy(h_hbms[k].at[sidx.at[b]], gbuf.at[0])
                pltpu.sync_copy(gbuf.at[0], acc.at[didx.at[b]], add=True)
                return carry

            for tid in range(NS):
                @pl.when(s == tid)
                def _(k=k):
                    lax.fori_loop(0, BT, bat, 0)
                plsc.subcore_barrier()
            plsc.subcore_barrier()
            _acc_writeback(acc, out_hbm.at[k], c, s)
            plsc.subcore_barrier()

    return pl.kernel(
        body,
        out_type=jax.ShapeDtypeStruct((K, NPAD, 128), jnp.float32),
        mesh=_mesh,
        scratch_types=scratch,
    )


_SC_SEG = {1: _make_sc_segment(1), 2: _make_sc_segment(2)}


def _row_mask(h, i):
    rid = i * h.shape[0] + lax.broadcasted_iota(jnp.int32, (h.shape[0], 1), 0)
    return jnp.where(rid < N, h, 0.0)


def _tc_first(degp, xpad, W1):
    def body(dp_ref, x_ref, w_ref, h_ref, dv_ref):
        deg = dp_ref[0, :, 0:1] + dp_ref[1, :, 0:1] + 1.0
        dv = lax.rsqrt(deg)
        pts = x_ref[:, 0:2]
        h = jnp.dot(pts, w_ref[...], preferred_element_type=jnp.float32) * dv
        h = jnp.concatenate([h, jnp.zeros((R_TC, 64), jnp.float32)], axis=1)
        h_ref[...] = _row_mask(h, pl.program_id(0))
        dv_ref[...] = dv

    return pl.pallas_call(
        body,
        grid=(NPAD // R_TC,),
        in_specs=[
            pl.BlockSpec((NC, R_TC, 16), lambda i: (0, i, 0)),
            pl.BlockSpec((R_TC, 3), lambda i: (i, 0)),
            pl.BlockSpec((2, 64), lambda i: (0, 0)),
        ],
        out_specs=[
            pl.BlockSpec((R_TC, 128), lambda i: (i, 0)),
            pl.BlockSpec((R_TC, 1), lambda i: (i, 0)),
        ],
        out_shape=[
            jax.ShapeDtypeStruct((NPAD, 128), jnp.float32),
            jax.ShapeDtypeStruct((NPAD, 1), jnp.float32),
        ],
    )(degp, xpad, W1)


def _make_tc_mid(dout, dnext):
    Kin = (dout + 127) // 128
    Knext = (dnext + 127) // 128

    def body(*refs):
        p_ref = refs[0]
        dv_ref, at_ref, b_ref, w_ref = refs[1:5]
        out_refs = refs[5:]
        parts = [p_ref[k] for k in range(Kin)]
        agg = parts[0] if Kin == 1 else jnp.concatenate(parts, axis=1)
        agg = agg[:, :dout]
        dv = dv_ref[...]
        hnew = jnp.maximum((agg * dv + b_ref[...]) * at_ref[...], 0.0)
        hp = jnp.dot(hnew, w_ref[...],
                     preferred_element_type=jnp.float32) * dv
        if dnext < Knext * 128:
            hp = jnp.concatenate(
                [hp, jnp.zeros((R_TC, Knext * 128 - dnext), jnp.float32)],
                axis=1)
        hp = _row_mask(hp, pl.program_id(0))
        for k in range(Knext):
            out_refs[k][...] = hp[:, k * 128:(k + 1) * 128]

    return pl.pallas_call(
        body,
        grid=(NPAD // R_TC,),
        in_specs=[
            pl.BlockSpec((Kin, R_TC, 128), lambda i: (0, i, 0)),
            pl.BlockSpec((R_TC, 1), lambda i: (i, 0)),
            pl.BlockSpec((R_TC, 1), lambda i: (i, 0)),
            pl.BlockSpec((1, dout), lambda i: (0, 0)),
            pl.BlockSpec((dout, dnext), lambda i: (0, 0)),
        ],
        out_specs=[pl.BlockSpec((R_TC, 128), lambda i: (i, 0))] * Knext,
        out_shape=[jax.ShapeDtypeStruct((NPAD, 128), jnp.float32)] * Knext,
    )


_TC_MID = [_make_tc_mid(DOUTS[l], DOUTS[l + 1]) for l in range(8)]


def _pool_matrix():
    P = np.zeros((6, 17), np.float32)
    P[0, :] = 1.0 / 17.0
    P[1, :11] = 1.0 / 11.0
    P[2, 11:] = 1.0 / 6.0
    P[3, [0, 1, 2, 3, 4]] = 1.0 / 5.0
    P[4, [5, 7, 9, 12, 14, 16]] = 1.0 / 6.0
    P[5, [6, 8, 10, 11, 13, 15]] = 1.0 / 6.0
    M = np.zeros((48, R_CD), np.float32)
    for i in range(6):
        for g in range(8):
            M[i * 8 + g, g * 17:(g + 1) * 17] = P[i]
    return jnp.asarray(M)


def _tc_pool(p9, dinv, att, b9, Mc, fcwt, fcb, ginv, beta):
    def body(p_ref, dv_ref, at_ref, b_ref, m_ref,
             w_ref, fb_ref, g_ref, bt_ref, out_ref):
        agg = jnp.concatenate([p_ref[0], p_ref[1]], axis=1)
        h9 = jnp.maximum((agg * dv_ref[...] + b_ref[...]) * at_ref[...], 0.0)
        h9 = _row_mask(h9, pl.program_id(0))
        pooled = jnp.dot(m_ref[...], h9, preferred_element_type=jnp.float32)
        feats = []
        for i in range(6):
            ti = jnp.dot(pooled[i * 8:(i + 1) * 8], w_ref[i],
                         preferred_element_type=jnp.float32) + fb_ref[i]
            feats.append(ti * g_ref[i] + bt_ref[i])
        row = jnp.concatenate(feats, axis=1)
        nrm = jnp.sqrt(jnp.sum(row * row, axis=1, keepdims=True))
        out_ref[...] = row / jnp.maximum(nrm, 1e-12)

    return pl.pallas_call(
        body,
        grid=(NPAD // R_CD,),
        in_specs=[
            pl.BlockSpec((2, R_CD, 128), lambda i: (0, i, 0)),
            pl.BlockSpec((R_CD, 1), lambda i: (i, 0)),
            pl.BlockSpec((R_CD, 1), lambda i: (i, 0)),
            pl.BlockSpec((1, 256), lambda i: (0, 0)),
            pl.BlockSpec((48, R_CD), lambda i: (0, 0)),
            pl.BlockSpec((6, 256, 256), lambda i: (0, 0, 0)),
            pl.BlockSpec((6, 1, 256), lambda i: (0, 0, 0)),
            pl.BlockSpec((6, 1, 256), lambda i: (0, 0, 0)),
            pl.BlockSpec((6, 1, 256), lambda i: (0, 0, 0)),
        ],
        out_specs=pl.BlockSpec((8, 1536), lambda i: (i, 0)),
        out_shape=jax.ShapeDtypeStruct((NPAD // R_CD * 8, 1536), jnp.float32),
    )(p9, dinv, att, b9, Mc, fcwt, fcb, ginv, beta)


def kernel(x, edge_index, W1, b1, W2, b2, W3, b3, W4, b4, W5, b5, W6, b6,
           W7, b7, W8, b8, W9, b9, fc_w, fc_b, gamma, beta):
    Ws = [W1, W2, W3, W4, W5, W6, W7, W8, W9]
    bs = [b1, b2, b3, b4, b5, b6, b7, b8, b9]

    xpad = jnp.pad(x, ((0, NPAD - N), (0, 0)))
    att = xpad[:, 2:3]
    pad = EPAD - E
    src3 = jnp.concatenate(
        [edge_index[0], jnp.full((pad,), N, jnp.int32)]).reshape(NW, BT, 128)
    dst3 = jnp.concatenate(
        [edge_index[1], jnp.full((pad,), N, jnp.int32)]).reshape(NW, BT, 128)

    degp = _sc_degree(dst3)
    h0, dinv = _tc_first(degp, xpad, W1)
    hch = [h0]

    for l in range(8):
        K = (DOUTS[l] + 127) // 128
        p = _SC_SEG[K](src3, dst3, *hch)
        outs = _TC_MID[l](p, dinv, att, bs[l].reshape(1, -1), Ws[l + 1])
        hch = list(outs)

    p9 = _SC_SEG[2](src3, dst3, *hch)
    fcwt = fc_w.transpose(0, 2, 1)
    ginv = (gamma / np.sqrt(1.0 + 1e-5)).reshape(6, 1, 256)
    outpad = _tc_pool(p9, dinv, att, b9.reshape(1, 256),
                      _pool_matrix(), fcwt, fc_b.reshape(6, 1, 256), ginv,
                      beta.reshape(6, 1, 256))
    return outpad[:N // 17]


# speedup vs baseline: 7.7048x; 7.7048x over previous
import functools

import jax
import jax.numpy as jnp
import numpy as np
from jax import lax
from jax.experimental import pallas as pl
from jax.experimental.pallas import tpu as pltpu
from jax.experimental.pallas import tpu_sc as plsc

N = 10013
E = 320416
NPAD = 10880
NC, NS = 2, 16
NW = NC * NS
H = NPAD // 2
ROWS = NPAD // NS
BT = (E // NW + 127) // 128
BT1 = BT + 1
EPAD = NW * BT * 128
DOUTS = [64, 64, 64, 128, 128, 128, 256, 256, 256]
R_TC = 640
R_CD = 136

_mesh = plsc.VectorSubcoreMesh(core_axis_name="c", subcore_axis_name="s")


def _zero_fill(ref, rows, cols):
    z = jnp.zeros((16,), jnp.float32)
    per_row = cols // 16

    def body(t, carry):
        ref[t // per_row, pl.ds((t % per_row) * 16, 16)] = z
        return carry

    lax.fori_loop(0, rows * per_row, body, 0, unroll=8)


@functools.partial(
    pl.kernel,
    out_type=jax.ShapeDtypeStruct((NC, NPAD, 16), jnp.float32),
    mesh=_mesh,
    scratch_types=[
        pltpu.VMEM((BT, 128), jnp.int32),
        pltpu.VMEM((128, 16), jnp.float32),
        pltpu.VMEM((128, 16), jnp.float32),
        pltpu.VMEM((128,), jnp.int32),
        pltpu.VMEM_SHARED((NPAD, 16), jnp.float32),
    ],
)
def _sc_degree(dst_hbm, out_hbm, didx, obuf, zbuf, ibuf, acc):
    c = lax.axis_index("c")
    s = lax.axis_index("s")
    wid = s * NC + c
    base = s * ROWS
    pltpu.sync_copy(dst_hbm.at[wid], didx)
    one = jnp.ones((16,), jnp.float32)

    def fill(t, carry):
        obuf[t, pl.ds(0, 16)] = one
        return carry

    lax.fori_loop(0, 128, fill, 0, unroll=8)
    _zero_fill(zbuf, 128, 16)
    off = 0
    while off < ROWS:
        sz = min(128, ROWS - off)
        pltpu.sync_copy(zbuf.at[pl.ds(0, sz)], acc.at[pl.ds(base + off, sz)])
        off += sz
    plsc.subcore_barrier()

    def bat(b, carry):
        for j in range(8):
            ibuf[pl.ds(j * 16, 16)] = didx[b, pl.ds(j * 16, 16)]
        pltpu.sync_copy(obuf, acc.at[ibuf], add=True)
        return carry

    lax.fori_loop(0, BT, bat, 0)
    plsc.subcore_barrier()
    pltpu.sync_copy(acc.at[pl.ds(base, ROWS)], out_hbm.at[c, pl.ds(base, ROWS)])


def _acc_init(h_hbm, acc, c, s):
    @pl.when(s < NS - 1)
    def _():
        pltpu.sync_copy(h_hbm.at[pl.ds(c * H + s * 344, 344)],
                        acc.at[pl.ds(s * 344, 344)])

    @pl.when(s == NS - 1)
    def _():
        pltpu.sync_copy(h_hbm.at[pl.ds(c * H + 5160, 280)],
                        acc.at[pl.ds(5160, 280)])


def _acc_writeback(acc, out_hbm, c, s):
    @pl.when(s < NS - 1)
    def _():
        pltpu.sync_copy(acc.at[pl.ds(s * 344, 344)],
                        out_hbm.at[pl.ds(c * H + s * 344, 344)])

    @pl.when(s == NS - 1)
    def _():
        pltpu.sync_copy(acc.at[pl.ds(5160, 280)],
                        out_hbm.at[pl.ds(c * H + 5160, 280)])


HSPR = 64
SSPR = 512


def _make_sc_segment(K):
    scratch = [
        pltpu.VMEM((BT, 128), jnp.int32),
        pltpu.VMEM((BT, 128), jnp.int32),
        pltpu.VMEM((4, 128, 128), jnp.float32),
        pltpu.VMEM((128,), jnp.int32),
        pltpu.VMEM((128,), jnp.int32),
        pltpu.VMEM((128,), jnp.int32),
        pltpu.VMEM((128,), jnp.int32),
        pltpu.VMEM_SHARED((H + HSPR, 128), jnp.float32),
        pltpu.SemaphoreType.DMA,
        pltpu.SemaphoreType.DMA,
        pltpu.SemaphoreType.DMA,
        pltpu.SemaphoreType.DMA,
    ]

    def body(src_hbm, dst_hbm, *rest):
        h_hbms = rest[:K]
        out_hbm = rest[K]
        sidx, didx, gbuf, ib0, ib1, ib2, ib3, acc = rest[K + 1:K + 9]
        gsems = rest[K + 9:]
        ibufs = (ib0, ib1, ib2, ib3)
        c = lax.axis_index("c")
        s = lax.axis_index("s")
        wid = s * NC + c
        pltpu.sync_copy(src_hbm.at[wid], sidx)
        pltpu.sync_copy(dst_hbm.at[wid], didx)
        iota16 = lax.iota(jnp.int32, 16)
        hvec = jnp.full((16,), H, jnp.int32)

        def make_rewrite(lo):
            def rewrite(t, carry):
                i = t // 8
                j = t % 8
                svec = sidx[i, pl.ds(j * 16, 16)]
                dvec = didx[i, pl.ds(j * 16, 16)]
                loc = dvec - lo
                inr = lax.shift_right_logical(loc - hvec, 31) * (
                    1 - lax.shift_right_logical(loc, 31))
                lane = t * 16 + iota16
                nsrc = N + (lane & (SSPR - 1))
                ndst = H + (lane & (HSPR - 1))
                out = 1 - inr
                sidx[i, pl.ds(j * 16, 16)] = inr * svec + out * nsrc
                didx[i, pl.ds(j * 16, 16)] = inr * loc + out * ndst
                return carry
            return rewrite

        @pl.when(c == 0)
        def _():
            lax.fori_loop(0, BT * 8, make_rewrite(0), 0)

        @pl.when(c == 1)
        def _():
            lax.fori_loop(0, BT * 8, make_rewrite(H), 0)

        for k in range(K):
            _acc_init(h_hbms[k], acc, c, s)
            plsc.subcore_barrier()
            def bat(b, carry, k=k):
                pltpu.sync_copy(h_hbms[k].at[sidx.at[b]], gbuf.at[0])
                for j in range(8):
                    ib0[pl.ds(j * 16, 16)] = didx[b, pl.ds(j * 16, 16)]
                pltpu.sync_copy(gbuf.at[0], acc.at[ib0], add=True)
                return carry

            lax.fori_loop(0, BT, bat, 0)
            plsc.subcore_barrier()
            _acc_writeback(acc, out_hbm.at[k], c, s)
            plsc.subcore_barrier()

    return pl.kernel(
        body,
        out_type=jax.ShapeDtypeStruct((K, NPAD, 128), jnp.float32),
        mesh=_mesh,
        scratch_types=scratch,
    )


_SC_SEG = {1: _make_sc_segment(1), 2: _make_sc_segment(2)}


def _row_mask(h, i):
    rid = i * h.shape[0] + lax.broadcasted_iota(jnp.int32, (h.shape[0], 1), 0)
    return jnp.where(rid < N, h, 0.0)


def _tc_first(degp, xpad, W1):
    def body(dp_ref, x_ref, w_ref, h_ref, dv_ref):
        deg = dp_ref[0, :, 0:1] + dp_ref[1, :, 0:1] + 1.0
        dv = lax.rsqrt(deg)
        pts = x_ref[:, 0:2]
        h = jnp.dot(pts, w_ref[...], preferred_element_type=jnp.float32) * dv
        h = jnp.concatenate([h, jnp.zeros((R_TC, 64), jnp.float32)], axis=1)
        h_ref[...] = _row_mask(h, pl.program_id(0))
        dv_ref[...] = dv

    return pl.pallas_call(
        body,
        grid=(NPAD // R_TC,),
        in_specs=[
            pl.BlockSpec((NC, R_TC, 16), lambda i: (0, i, 0)),
            pl.BlockSpec((R_TC, 3), lambda i: (i, 0)),
            pl.BlockSpec((2, 64), lambda i: (0, 0)),
        ],
        out_specs=[
            pl.BlockSpec((R_TC, 128), lambda i: (i, 0)),
            pl.BlockSpec((R_TC, 1), lambda i: (i, 0)),
        ],
        out_shape=[
            jax.ShapeDtypeStruct((NPAD, 128), jnp.float32),
            jax.ShapeDtypeStruct((NPAD, 1), jnp.float32),
        ],
    )(degp, xpad, W1)


def _make_tc_mid(dout, dnext):
    Kin = (dout + 127) // 128
    Knext = (dnext + 127) // 128

    def body(*refs):
        p_ref = refs[0]
        dv_ref, at_ref, b_ref, w_ref = refs[1:5]
        out_refs = refs[5:]
        parts = [p_ref[k] for k in range(Kin)]
        agg = parts[0] if Kin == 1 else jnp.concatenate(parts, axis=1)
        agg = agg[:, :dout]
        dv = dv_ref[...]
        hnew = jnp.maximum((agg * dv + b_ref[...]) * at_ref[...], 0.0)
        hp = jnp.dot(hnew, w_ref[...],
                     preferred_element_type=jnp.float32) * dv
        if dnext < Knext * 128:
            hp = jnp.concatenate(
                [hp, jnp.zeros((R_TC, Knext * 128 - dnext), jnp.float32)],
                axis=1)
        hp = _row_mask(hp, pl.program_id(0))
        for k in range(Knext):
            out_refs[k][...] = hp[:, k * 128:(k + 1) * 128]

    return pl.pallas_call(
        body,
        grid=(NPAD // R_TC,),
        in_specs=[
            pl.BlockSpec((Kin, R_TC, 128), lambda i: (0, i, 0)),
            pl.BlockSpec((R_TC, 1), lambda i: (i, 0)),
            pl.BlockSpec((R_TC, 1), lambda i: (i, 0)),
            pl.BlockSpec((1, dout), lambda i: (0, 0)),
            pl.BlockSpec((dout, dnext), lambda i: (0, 0)),
        ],
        out_specs=[pl.BlockSpec((R_TC, 128), lambda i: (i, 0))] * Knext,
        out_shape=[jax.ShapeDtypeStruct((NPAD, 128), jnp.float32)] * Knext,
    )


_TC_MID = [_make_tc_mid(DOUTS[l], DOUTS[l + 1]) for l in range(8)]


def _pool_matrix():
    P = np.zeros((6, 17), np.float32)
    P[0, :] = 1.0 / 17.0
    P[1, :11] = 1.0 / 11.0
    P[2, 11:] = 1.0 / 6.0
    P[3, [0, 1, 2, 3, 4]] = 1.0 / 5.0
    P[4, [5, 7, 9, 12, 14, 16]] = 1.0 / 6.0
    P[5, [6, 8, 10, 11, 13, 15]] = 1.0 / 6.0
    M = np.zeros((48, R_CD), np.float32)
    for i in range(6):
        for g in range(8):
            M[i * 8 + g, g * 17:(g + 1) * 17] = P[i]
    return jnp.asarray(M)


def _tc_pool(p9, dinv, att, b9, Mc, fcwt, fcb, ginv, beta):
    def body(p_ref, dv_ref, at_ref, b_ref, m_ref,
             w_ref, fb_ref, g_ref, bt_ref, out_ref):
        agg = jnp.concatenate([p_ref[0], p_ref[1]], axis=1)
        h9 = jnp.maximum((agg * dv_ref[...] + b_ref[...]) * at_ref[...], 0.0)
        h9 = _row_mask(h9, pl.program_id(0))
        pooled = jnp.dot(m_ref[...], h9, preferred_element_type=jnp.float32)
        feats = []
        for i in range(6):
            ti = jnp.dot(pooled[i * 8:(i + 1) * 8], w_ref[i],
                         preferred_element_type=jnp.float32) + fb_ref[i]
            feats.append(ti * g_ref[i] + bt_ref[i])
        row = jnp.concatenate(feats, axis=1)
        nrm = jnp.sqrt(jnp.sum(row * row, axis=1, keepdims=True))
        out_ref[...] = row / jnp.maximum(nrm, 1e-12)

    return pl.pallas_call(
        body,
        grid=(NPAD // R_CD,),
        in_specs=[
            pl.BlockSpec((2, R_CD, 128), lambda i: (0, i, 0)),
            pl.BlockSpec((R_CD, 1), lambda i: (i, 0)),
            pl.BlockSpec((R_CD, 1), lambda i: (i, 0)),
            pl.BlockSpec((1, 256), lambda i: (0, 0)),
            pl.BlockSpec((48, R_CD), lambda i: (0, 0)),
            pl.BlockSpec((6, 256, 256), lambda i: (0, 0, 0)),
            pl.BlockSpec((6, 1, 256), lambda i: (0, 0, 0)),
            pl.BlockSpec((6, 1, 256), lambda i: (0, 0, 0)),
            pl.BlockSpec((6, 1, 256), lambda i: (0, 0, 0)),
        ],
        out_specs=pl.BlockSpec((8, 1536), lambda i: (i, 0)),
        out_shape=jax.ShapeDtypeStruct((NPAD // R_CD * 8, 1536), jnp.float32),
    )(p9, dinv, att, b9, Mc, fcwt, fcb, ginv, beta)


def kernel(x, edge_index, W1, b1, W2, b2, W3, b3, W4, b4, W5, b5, W6, b6,
           W7, b7, W8, b8, W9, b9, fc_w, fc_b, gamma, beta):
    Ws = [W1, W2, W3, W4, W5, W6, W7, W8, W9]
    bs = [b1, b2, b3, b4, b5, b6, b7, b8, b9]

    xpad = jnp.pad(x, ((0, NPAD - N), (0, 0)))
    att = xpad[:, 2:3]
    pad = EPAD - E
    src3 = jnp.concatenate(
        [edge_index[0], jnp.full((pad,), N, jnp.int32)]).reshape(NW, BT, 128)
    dst3 = jnp.concatenate(
        [edge_index[1], jnp.full((pad,), N, jnp.int32)]).reshape(NW, BT, 128)

    degp = _sc_degree(dst3)
    h0, dinv = _tc_first(degp, xpad, W1)
    hch = [h0]

    for l in range(8):
        K = (DOUTS[l] + 127) // 128
        p = _SC_SEG[K](src3, dst3, *hch)
        outs = _TC_MID[l](p, dinv, att, bs[l].reshape(1, -1), Ws[l + 1])
        hch = list(outs)

    p9 = _SC_SEG[2](src3, dst3, *hch)
    fcwt = fc_w.transpose(0, 2, 1)
    ginv = (gamma / np.sqrt(1.0 + 1e-5)).reshape(6, 1, 256)
    outpad = _tc_pool(p9, dinv, att, b9.reshape(1, 256),
                      _pool_matrix(), fcwt, fc_b.reshape(6, 1, 256), ginv,
                      beta.reshape(6, 1, 256))
    return outpad[:N // 17]
